# Initial kernel scaffold; baseline (speedup 1.0000x reference)
import jax
import jax.numpy as jnp
from jax.experimental import pallas as pl


def kernel(q, k, v, lsh_projection):
    raise NotImplementedError("write your pallas kernel here")


# speedup vs baseline: 1.5330x; 1.5330x over previous
import jax
import jax.numpy as jnp
from jax.experimental import pallas as pl


def _attn_body(q_ref, k_ref, v_ref, o_ref):
    q = q_ref[0]
    k64 = k_ref[0]
    v64 = v_ref[0]
    s = jax.lax.dot_general(
        q, k64, (((1,), (1,)), ((), ())),
        preferred_element_type=jnp.float32,
        precision=jax.lax.Precision.DEFAULT,
    )
    m = jnp.max(s, axis=-1, keepdims=True)
    e = jnp.exp(s - m)
    p = e / jnp.sum(e, axis=-1, keepdims=True)
    o_ref[0] = jax.lax.dot_general(
        p, v64, (((1,), (0,)), ((), ())),
        preferred_element_type=jnp.float32,
        precision=jax.lax.Precision.DEFAULT,
    )


def kernel(q, k, v, lsh_projection):
    B, N, D = q.shape
    H = lsh_projection.shape[0]
    k64 = k[:, :H, :]
    v64 = v[:, :H, :]
    TN = 2048
    return pl.pallas_call(
        _attn_body,
        grid=(B, N // TN),
        in_specs=[
            pl.BlockSpec((1, TN, D), lambda b, i: (b, i, 0)),
            pl.BlockSpec((1, H, D), lambda b, i: (b, 0, 0)),
            pl.BlockSpec((1, H, D), lambda b, i: (b, 0, 0)),
        ],
        out_specs=pl.BlockSpec((1, TN, D), lambda b, i: (b, i, 0)),
        out_shape=jax.ShapeDtypeStruct((B, N, D), jnp.float32),
    )(q, k64, v64)
